# baseline (speedup 1.0000x reference)
import functools

import jax
import jax.numpy as jnp
from jax import lax
from jax.experimental import pallas as pl
from jax.experimental.pallas import tpu as pltpu
from jax.experimental.pallas import tpu_sc as plsc

NB = 50
K = 32
D = 128
DM = 256
ROWS = 2176
RPW = ROWS // 32
W1 = 56
NEG = -3.0e38


def _rsqrt_newton(x):
    xi = plsc.bitcast(x, jnp.int32)
    yi = 0x5F3759DF - lax.shift_right_logical(xi, 1)
    y = plsc.bitcast(yi, jnp.float32)
    for _ in range(3):
        y = y * (1.5 - 0.5 * x * y * y)
    return y


def _sort16(x):
    return jnp.sort(x)


def _merge2(a, b):
    rb = jnp.flip(b, 0)
    lo = jnp.minimum(a, rb)
    hi = jnp.maximum(a, rb)
    return _sort16(lo), _sort16(hi)


def _bmerge32(p, q):
    lo = jnp.minimum(p, q)
    hi = jnp.maximum(p, q)
    return _sort16(lo), _sort16(hi)


def _sc_encode(table, idx1):
    mesh = plsc.VectorSubcoreMesh(core_axis_name="c", subcore_axis_name="s")

    @functools.partial(
        pl.kernel, mesh=mesh,
        compiler_params=pltpu.CompilerParams(needs_layout_passes=False),
        out_type=jax.ShapeDtypeStruct((32, RPW, 2 * D), jnp.float32),
        scratch_types=(
            [pltpu.VMEM((RPW, 128), jnp.int32)]
            + [pltpu.VMEM((128, D), jnp.float32) for _ in range(4)]
            + [pltpu.VMEM((64,), jnp.float32),
               pltpu.VMEM((RPW, 2 * D), jnp.float32)]
            + [pltpu.SemaphoreType.DMA for _ in range(6)]
        ),
    )
    def k(table_h, idx_h, mean_o,
          idx_v, buf0, buf1, buf2, buf3,
          w_buf, out_v, psem, g0, g1, g2, g3, wsem):
        nc = 2
        wid = lax.axis_index("s") * nc + lax.axis_index("c")
        bufs = (buf0, buf1, buf2, buf3)
        gs = (g0, g1, g2, g3)

        pltpu.async_copy(idx_h.at[wid], idx_v, psem).wait()

        iota = lax.iota(jnp.int32, 16)
        valid3 = iota < (NB - 48)
        c32 = jnp.full((16,), K, jnp.int32)
        jrow = [jnp.where((g * 16 + iota) < NB, 1 + g * 16 + iota, 0)
                for g in range(4)]
        zrow = jnp.zeros((16,), jnp.int32)
        zero = jnp.zeros((16,), jnp.float32)

        def issue(r, b):
            pltpu.async_copy(table_h.at[idx_v.at[r]], bufs[b], gs[b])

        for b in range(4):
            issue(b, b)

        def do_row(r, b):
            ab = bufs[b]
            pltpu.make_async_copy(
                table_h.at[idx_v.at[r]], ab, gs[b]).wait()

            def fbody(f8, carry):
                accs = list(carry)
                for i in range(8):
                    fv = jnp.full((16,), i, jnp.int32) + f8 * 8
                    cf = plsc.load_gather(ab, [zrow, fv])
                    for g in range(4):
                        col = plsc.load_gather(ab, [jrow[g], fv])
                        accs[2 * g] = accs[2 * g] + cf * col
                        accs[2 * g + 1] = accs[2 * g + 1] + col * col
                return tuple(accs)

            accs = lax.fori_loop(0, 16, fbody, (zero,) * 8)
            keys = []
            for g in range(4):
                d_g, n_g = accs[2 * g], accs[2 * g + 1]
                kg = d_g * _rsqrt_newton(jnp.maximum(n_g, 1e-16))
                if g == 3:
                    kg = jnp.where(valid3, kg, NEG)
                keys.append(kg)

            s0, s1, s2, s3 = (_sort16(x) for x in keys)
            a0, a1 = _merge2(s0, s1)
            b0, b1 = _merge2(s2, s3)
            ry0, ry1 = jnp.flip(b1, 0), jnp.flip(b0, 0)
            h0 = jnp.maximum(a0, ry0)
            h1 = jnp.maximum(a1, ry1)
            z2, _ = _bmerge32(h0, h1)
            t_thr = jnp.broadcast_to(jnp.min(z2), (16,))

            gts = [kg > t_thr for kg in keys]
            c_gt = jnp.zeros((16,), jnp.int32)
            for g in range(4):
                c_gt = c_gt + plsc.all_reduce_population_count(gts[g])
            allow = c32 - c_gt
            prior = jnp.zeros((16,), jnp.int32)
            for g in range(4):
                eq = keys[g] == t_thr
                inc = plsc.cumsum(eq.astype(jnp.int32))
                take = eq & ((inc + prior) <= allow)
                prior = prior + plsc.all_reduce_population_count(eq)
                sel = gts[g] | take
                w_buf[16 * g:16 * g + 16] = sel.astype(jnp.float32)

            def wbody(j5, carry):
                accs = list(carry)
                for i in range(5):
                    wj = plsc.load_gather(
                        w_buf, [jnp.full((16,), i, jnp.int32) + j5 * 5])
                    for c in range(8):
                        accs[c] = accs[c] + wj * ab[1 + j5 * 5 + i,
                                                    16 * c:16 * c + 16]
                        accs[8 + c] = accs[8 + c] + wj * ab[
                            1 + NB + j5 * 5 + i, 16 * c:16 * c + 16]
                return tuple(accs)

            accs2 = lax.fori_loop(0, 10, wbody, (zero,) * 16)
            for c in range(8):
                out_v[r, D + 16 * c:D + 16 * c + 16] = accs2[c] * (1.0 / K)
                out_v[r, 16 * c:16 * c + 16] = accs2[8 + c] * (1.0 / K)

            @pl.when(r + 4 < RPW)
            def _():
                issue(r + 4, b)

        def quad(t, carry):
            for b in range(4):
                do_row(4 * t + b, b)
            return carry

        lax.fori_loop(0, RPW // 4, quad, 0)
        pltpu.async_copy(out_v, mean_o.at[wid], wsem).wait()

    return k(table, idx1.reshape(32, RPW, 128)).reshape(ROWS, 2 * D)


def _head_kernel(mean_ref, gcnW_ref, gcnb_ref, p1W_ref, p1b_ref, p2W_ref,
                 p2b_ref, ln_g_ref, ln_b_ref, Wih_ref, Whh_ref, bih_ref,
                 bhh_ref, out_ref, B):
    mc = mean_ref[...]
    neigh = jnp.tanh(jnp.dot(mc, gcnW_ref[...].T,
                             preferred_element_type=jnp.float32)
                     + gcnb_ref[...])
    qn = jnp.concatenate([neigh[0:B], neigh[B:2 * B]], axis=1)
    sn = jnp.concatenate([neigh[2 * B:2 * B + 5],
                          neigh[2 * B + 8:2 * B + 13]], axis=1)

    p1W = p1W_ref[...]
    p2W = p2W_ref[...]
    ln_g = ln_g_ref[...]
    ln_b = ln_b_ref[...]

    def enc(x):
        out = jax.nn.relu(jnp.dot(x, p1W.T, preferred_element_type=jnp.float32)
                          + p1b_ref[...])
        out = jnp.dot(out, p2W.T, preferred_element_type=jnp.float32) + p2b_ref[...]
        out = out + x
        m = jnp.mean(out, axis=-1, keepdims=True)
        v = jnp.mean((out - m) ** 2, axis=-1, keepdims=True)
        return (out - m) / jnp.sqrt(v + 1e-5) * ln_g + ln_b

    support_g = jnp.mean(enc(sn), axis=0, keepdims=True)
    query_g = enc(qn)

    Wih = Wih_ref[...]
    Whh = Whh_ref[...]
    bih = bih_ref[...]
    bhh = bhh_ref[...]
    sup_b = jnp.broadcast_to(support_g, (B, DM))

    h_r = jnp.zeros((B, 2 * DM), jnp.float32)
    c = jnp.zeros((B, 2 * DM), jnp.float32)
    h = query_g
    for _ in range(2):
        gates = (jnp.dot(query_g, Wih.T, preferred_element_type=jnp.float32)
                 + bih
                 + jnp.dot(h_r, Whh.T, preferred_element_type=jnp.float32)
                 + bhh)
        i_g = gates[:, 0:2 * DM]
        f_g = gates[:, 2 * DM:4 * DM]
        g_g = gates[:, 4 * DM:6 * DM]
        o_g = gates[:, 6 * DM:8 * DM]
        c = jax.nn.sigmoid(f_g) * c + jax.nn.sigmoid(i_g) * jnp.tanh(g_g)
        h_new = jax.nn.sigmoid(o_g) * jnp.tanh(c)
        h = query_g + h_new[:, :DM]
        h_r = jnp.concatenate([h, sup_b], axis=1)

    qf = h / jnp.maximum(jnp.linalg.norm(h, axis=-1, keepdims=True), 1e-12)
    sv = support_g[0]
    sv = sv / jnp.maximum(jnp.linalg.norm(sv), 1e-12)
    out_ref[...] = jnp.dot(qf, sv[:, None],
                           preferred_element_type=jnp.float32)[:, 0]


def kernel(query, support, q_l_conn, q_l_deg, q_r_conn, q_r_deg,
           s_l_conn, s_l_deg, s_r_conn, s_r_deg, table,
           gcn_wW, gcn_wb, gcn_b, p1W, p1b, p2W, p2b, ln_g, ln_b,
           Wih, Whh, bih, bhh):
    B = query.shape[0]
    FEW = support.shape[0]

    ids = jnp.zeros((ROWS,), jnp.int32)
    ids = ids.at[0:B].set(query[:, 0].astype(jnp.int32))
    ids = ids.at[B:2 * B].set(query[:, 1].astype(jnp.int32))
    ids = ids.at[2 * B:2 * B + FEW].set(support[:, 0].astype(jnp.int32))
    ids = ids.at[2 * B + 8:2 * B + 8 + FEW].set(support[:, 1].astype(jnp.int32))
    conn = jnp.zeros((ROWS, NB, 2), jnp.int32)
    conn = conn.at[0:B].set(q_l_conn.astype(jnp.int32))
    conn = conn.at[B:2 * B].set(q_r_conn.astype(jnp.int32))
    conn = conn.at[2 * B:2 * B + FEW].set(s_l_conn.astype(jnp.int32))
    conn = conn.at[2 * B + 8:2 * B + 8 + FEW].set(s_r_conn.astype(jnp.int32))

    idx1 = jnp.concatenate(
        [ids[:, None], conn[:, :, 1], conn[:, :, 0],
         jnp.zeros((ROWS, 27), jnp.int32)], axis=1)

    mean = _sc_encode(table, idx1)

    scores = pl.pallas_call(
        functools.partial(_head_kernel, B=B),
        out_shape=jax.ShapeDtypeStruct((B,), jnp.float32),
        compiler_params=pltpu.CompilerParams(
            vmem_limit_bytes=63 * 1024 * 1024),
    )(mean, gcn_wW, gcn_wb + gcn_b, p1W, p1b, p2W, p2b, ln_g, ln_b,
      Wih, Whh, bih, bhh)
    return scores


# speedup vs baseline: 4.6758x; 4.6758x over previous
import functools

import jax
import jax.numpy as jnp
from jax import lax
from jax.experimental import pallas as pl
from jax.experimental.pallas import tpu as pltpu
from jax.experimental.pallas import tpu_sc as plsc

NB = 50
K = 32
D = 128
DM = 256
ROWS = 2176
RPW = ROWS // 32
W1 = 56
NEG = -3.0e38


def _rsqrt_newton(x):
    xi = plsc.bitcast(x, jnp.int32)
    yi = 0x5F3759DF - lax.shift_right_logical(xi, 1)
    y = plsc.bitcast(yi, jnp.float32)
    for _ in range(3):
        y = y * (1.5 - 0.5 * x * y * y)
    return y


def _sort16(x):
    return jnp.sort(x)


def _merge2(a, b):
    rb = jnp.flip(b, 0)
    lo = jnp.minimum(a, rb)
    hi = jnp.maximum(a, rb)
    return _sort16(lo), _sort16(hi)


def _bmerge32(p, q):
    lo = jnp.minimum(p, q)
    hi = jnp.maximum(p, q)
    return _sort16(lo), _sort16(hi)


def _sc_encode(table, idx1):
    mesh = plsc.VectorSubcoreMesh(core_axis_name="c", subcore_axis_name="s")

    @functools.partial(
        pl.kernel, mesh=mesh,
        compiler_params=pltpu.CompilerParams(needs_layout_passes=False),
        out_type=jax.ShapeDtypeStruct((32, RPW, 2 * D), jnp.float32),
        scratch_types=(
            [pltpu.VMEM((RPW, 128), jnp.int32)]
            + [pltpu.VMEM((128, D), jnp.float32) for _ in range(4)]
            + [pltpu.VMEM((64,), jnp.float32),
               pltpu.VMEM((RPW, 2 * D), jnp.float32)]
            + [pltpu.SemaphoreType.DMA for _ in range(6)]
        ),
    )
    def k(table_h, idx_h, mean_o,
          idx_v, buf0, buf1, buf2, buf3,
          w_buf, out_v, psem, g0, g1, g2, g3, wsem):
        nc = 2
        wid = lax.axis_index("s") * nc + lax.axis_index("c")
        bufs = (buf0, buf1, buf2, buf3)
        gs = (g0, g1, g2, g3)

        pltpu.async_copy(idx_h.at[wid], idx_v, psem).wait()

        iota = lax.iota(jnp.int32, 16)
        valid3 = iota < (NB - 48)
        c32 = jnp.full((16,), K, jnp.int32)
        jrow = [jnp.where((g * 16 + iota) < NB, 1 + g * 16 + iota, 0)
                for g in range(4)]
        zrow = jnp.zeros((16,), jnp.int32)
        zero = jnp.zeros((16,), jnp.float32)

        def issue(r, b):
            pltpu.async_copy(table_h.at[idx_v.at[r]], bufs[b], gs[b])

        for b in range(4):
            issue(b, b)

        def do_row(r, b):
            ab = bufs[b]
            pltpu.make_async_copy(
                table_h.at[idx_v.at[r]], ab, gs[b]).wait()

            def fbody(f8, carry):
                accs = list(carry)
                for i in range(8):
                    fv = jnp.full((16,), i, jnp.int32) + f8 * 8
                    cf = plsc.load_gather(ab, [zrow, fv])
                    for g in range(4):
                        col = plsc.load_gather(ab, [jrow[g], fv])
                        accs[2 * g] = accs[2 * g] + cf * col
                        accs[2 * g + 1] = accs[2 * g + 1] + col * col
                return tuple(accs)

            accs = lax.fori_loop(0, 16, fbody, (zero,) * 8)
            keys = []
            for g in range(4):
                d_g, n_g = accs[2 * g], accs[2 * g + 1]
                kg = d_g * _rsqrt_newton(jnp.maximum(n_g, 1e-16))
                if g == 3:
                    kg = jnp.where(valid3, kg, NEG)
                keys.append(kg)

            s0, s1, s2, s3 = (_sort16(x) for x in keys)
            a0, a1 = _merge2(s0, s1)
            b0, b1 = _merge2(s2, s3)
            ry0, ry1 = jnp.flip(b1, 0), jnp.flip(b0, 0)
            h0 = jnp.maximum(a0, ry0)
            h1 = jnp.maximum(a1, ry1)
            z2, _ = _bmerge32(h0, h1)
            t_thr = jnp.broadcast_to(jnp.min(z2), (16,))

            gts = [kg > t_thr for kg in keys]
            c_gt = jnp.zeros((16,), jnp.int32)
            for g in range(4):
                c_gt = c_gt + plsc.all_reduce_population_count(gts[g])
            allow = c32 - c_gt
            prior = jnp.zeros((16,), jnp.int32)
            for g in range(4):
                eq = keys[g] == t_thr
                inc = plsc.cumsum(eq.astype(jnp.int32))
                take = eq & ((inc + prior) <= allow)
                prior = prior + plsc.all_reduce_population_count(eq)
                sel = gts[g] | take
                w_buf[16 * g:16 * g + 16] = sel.astype(jnp.float32)

            def wbody(j5, carry):
                accs = list(carry)
                for i in range(5):
                    wj = plsc.load_gather(
                        w_buf, [jnp.full((16,), i, jnp.int32) + j5 * 5])
                    for c in range(8):
                        accs[c] = accs[c] + wj * ab[1 + j5 * 5 + i,
                                                    16 * c:16 * c + 16]
                        accs[8 + c] = accs[8 + c] + wj * ab[
                            1 + NB + j5 * 5 + i, 16 * c:16 * c + 16]
                return tuple(accs)

            accs2 = lax.fori_loop(0, 10, wbody, (zero,) * 16)
            for c in range(8):
                out_v[r, D + 16 * c:D + 16 * c + 16] = accs2[c] * (1.0 / K)
                out_v[r, 16 * c:16 * c + 16] = accs2[8 + c] * (1.0 / K)

            @pl.when(r + 4 < RPW)
            def _():
                issue(r + 4, b)

        def quad(t, carry):
            for b in range(4):
                do_row(4 * t + b, b)
            return carry

        lax.fori_loop(0, RPW // 4, quad, 0)
        pltpu.async_copy(out_v, mean_o.at[wid], wsem).wait()

    return k(table, idx1.reshape(32, RPW, 128)).reshape(ROWS, 2 * D)


def _head_kernel(mean_ref, gcnW_ref, gcnb_ref, p1W_ref, p1b_ref, p2W_ref,
                 p2b_ref, ln_g_ref, ln_b_ref, Wih_ref, Whh_ref, bih_ref,
                 bhh_ref, out_ref, B):
    mc = mean_ref[...]
    neigh = jnp.tanh(jnp.dot(mc, gcnW_ref[...].T,
                             preferred_element_type=jnp.float32)
                     + gcnb_ref[...])
    qn = jnp.concatenate([neigh[0:B], neigh[B:2 * B]], axis=1)
    sn = jnp.concatenate([neigh[2 * B:2 * B + 5],
                          neigh[2 * B + 8:2 * B + 13]], axis=1)

    p1W = p1W_ref[...]
    p2W = p2W_ref[...]
    ln_g = ln_g_ref[...]
    ln_b = ln_b_ref[...]

    def enc(x):
        out = jax.nn.relu(jnp.dot(x, p1W.T, preferred_element_type=jnp.float32)
                          + p1b_ref[...])
        out = jnp.dot(out, p2W.T, preferred_element_type=jnp.float32) + p2b_ref[...]
        out = out + x
        m = jnp.mean(out, axis=-1, keepdims=True)
        v = jnp.mean((out - m) ** 2, axis=-1, keepdims=True)
        return (out - m) / jnp.sqrt(v + 1e-5) * ln_g + ln_b

    support_g = jnp.mean(enc(sn), axis=0, keepdims=True)
    query_g = enc(qn)

    Wih = Wih_ref[...]
    Whh = Whh_ref[...]
    bih = bih_ref[...]
    bhh = bhh_ref[...]
    sup_b = jnp.broadcast_to(support_g, (B, DM))

    h_r = jnp.zeros((B, 2 * DM), jnp.float32)
    c = jnp.zeros((B, 2 * DM), jnp.float32)
    h = query_g
    for _ in range(2):
        gates = (jnp.dot(query_g, Wih.T, preferred_element_type=jnp.float32)
                 + bih
                 + jnp.dot(h_r, Whh.T, preferred_element_type=jnp.float32)
                 + bhh)
        i_g = gates[:, 0:2 * DM]
        f_g = gates[:, 2 * DM:4 * DM]
        g_g = gates[:, 4 * DM:6 * DM]
        o_g = gates[:, 6 * DM:8 * DM]
        c = jax.nn.sigmoid(f_g) * c + jax.nn.sigmoid(i_g) * jnp.tanh(g_g)
        h_new = jax.nn.sigmoid(o_g) * jnp.tanh(c)
        h = query_g + h_new[:, :DM]
        h_r = jnp.concatenate([h, sup_b], axis=1)

    qf = h / jnp.maximum(jnp.linalg.norm(h, axis=-1, keepdims=True), 1e-12)
    sv = support_g[0]
    sv = sv / jnp.maximum(jnp.linalg.norm(sv), 1e-12)
    out_ref[...] = jnp.dot(qf, sv[:, None],
                           preferred_element_type=jnp.float32)[:, 0]


def kernel(query, support, q_l_conn, q_l_deg, q_r_conn, q_r_deg,
           s_l_conn, s_l_deg, s_r_conn, s_r_deg, table,
           gcn_wW, gcn_wb, gcn_b, p1W, p1b, p2W, p2b, ln_g, ln_b,
           Wih, Whh, bih, bhh):
    B = query.shape[0]
    FEW = support.shape[0]

    ids = jnp.zeros((ROWS,), jnp.int32)
    ids = ids.at[0:B].set(query[:, 0].astype(jnp.int32))
    ids = ids.at[B:2 * B].set(query[:, 1].astype(jnp.int32))
    ids = ids.at[2 * B:2 * B + FEW].set(support[:, 0].astype(jnp.int32))
    ids = ids.at[2 * B + 8:2 * B + 8 + FEW].set(support[:, 1].astype(jnp.int32))
    conn = jnp.zeros((ROWS, NB, 2), jnp.int32)
    conn = conn.at[0:B].set(q_l_conn.astype(jnp.int32))
    conn = conn.at[B:2 * B].set(q_r_conn.astype(jnp.int32))
    conn = conn.at[2 * B:2 * B + FEW].set(s_l_conn.astype(jnp.int32))
    conn = conn.at[2 * B + 8:2 * B + 8 + FEW].set(s_r_conn.astype(jnp.int32))

    idx1 = jnp.concatenate(
        [ids[:, None], conn[:, :, 1], conn[:, :, 0],
         (jnp.arange(27, dtype=jnp.int32)[None, :]
          + 53 * jnp.arange(ROWS, dtype=jnp.int32)[:, None]) % 100000],
        axis=1)

    mean = _sc_encode(table, idx1)

    scores = pl.pallas_call(
        functools.partial(_head_kernel, B=B),
        out_shape=jax.ShapeDtypeStruct((B,), jnp.float32),
        compiler_params=pltpu.CompilerParams(
            vmem_limit_bytes=63 * 1024 * 1024),
    )(mean, gcn_wW, gcn_wb + gcn_b, p1W, p1b, p2W, p2b, ln_g, ln_b,
      Wih, Whh, bih, bhh)
    return scores
